# Initial kernel scaffold; baseline (speedup 1.0000x reference)
import jax
import jax.numpy as jnp
from jax.experimental import pallas as pl


def kernel(x, table):
    raise NotImplementedError("write your pallas kernel here")


# speedup vs baseline: 1.6683x; 1.6683x over previous
import jax
import jax.numpy as jnp
from jax.experimental import pallas as pl


_BS = 1024


def _add_kernel(x_ref, t_ref, o_ref):
    o_ref[...] = x_ref[...] + t_ref[...]


def kernel(x, table):
    batch, seq_len, dim = x.shape
    pos = table[:seq_len]
    grid = (seq_len // _BS, batch)
    return pl.pallas_call(
        _add_kernel,
        grid=grid,
        in_specs=[
            pl.BlockSpec((1, _BS, dim), lambda i, j: (j, i, 0)),
            pl.BlockSpec((_BS, dim), lambda i, j: (i, 0)),
        ],
        out_specs=pl.BlockSpec((1, _BS, dim), lambda i, j: (j, i, 0)),
        out_shape=jax.ShapeDtypeStruct((batch, seq_len, dim), x.dtype),
    )(x, pos)


# speedup vs baseline: 1.7356x; 1.0404x over previous
import jax
import jax.numpy as jnp
from jax.experimental import pallas as pl


_BS = 2048


def _add_kernel(x_ref, t_ref, o_ref):
    o_ref[...] = x_ref[...] + t_ref[...]


def kernel(x, table):
    batch, seq_len, dim = x.shape
    pos = table[:seq_len]
    grid = (seq_len // _BS, batch)
    return pl.pallas_call(
        _add_kernel,
        grid=grid,
        in_specs=[
            pl.BlockSpec((1, _BS, dim), lambda i, j: (j, i, 0)),
            pl.BlockSpec((_BS, dim), lambda i, j: (i, 0)),
        ],
        out_specs=pl.BlockSpec((1, _BS, dim), lambda i, j: (j, i, 0)),
        out_shape=jax.ShapeDtypeStruct((batch, seq_len, dim), x.dtype),
    )(x, pos)


# speedup vs baseline: 1.7390x; 1.0019x over previous
import jax
import jax.numpy as jnp
from jax.experimental import pallas as pl
from jax.experimental.pallas import tpu as pltpu


_BS = 2048


def _add_kernel(x_ref, t_ref, o_ref):
    o_ref[...] = x_ref[...] + t_ref[...]


def kernel(x, table):
    batch, seq_len, dim = x.shape
    pos = table[:seq_len]
    grid = (seq_len // _BS, batch)
    return pl.pallas_call(
        _add_kernel,
        grid=grid,
        in_specs=[
            pl.BlockSpec((1, _BS, dim), lambda i, j: (j, i, 0)),
            pl.BlockSpec((_BS, dim), lambda i, j: (i, 0)),
        ],
        out_specs=pl.BlockSpec((1, _BS, dim), lambda i, j: (j, i, 0)),
        out_shape=jax.ShapeDtypeStruct((batch, seq_len, dim), x.dtype),
        compiler_params=pltpu.CompilerParams(
            dimension_semantics=("parallel", "arbitrary"),
        ),
    )(x, pos)
